# baseline (speedup 1.0000x reference)
import jax
import jax.numpy as jnp
from jax import lax
from jax.experimental import pallas as pl
from jax.experimental.pallas import tpu as pltpu
from jax.experimental.pallas import tpu_sc as plsc

N = 10000
E = 320000
D = 128
NC = 2
NS = 16
NW = NC * NS
CHUNK = 128
CPW = 80
PADE = NW * CPW * CHUNK
NP_ = 10240
DUMMY = N
RPT = NP_ // NS
WB = 128
DEGW = 128
DEG_Q = 8
IDXB = 16

_MESH = plsc.VectorSubcoreMesh(
    core_axis_name="c", subcore_axis_name="s", num_cores=NC, num_subcores=NS
)


@pl.kernel(
    out_type=jax.ShapeDtypeStruct((NC, NP_, D), jnp.float32),
    mesh=_MESH,
    scratch_types=[
        pltpu.VMEM((IDXB, CHUNK), jnp.int32),
        pltpu.VMEM((IDXB, CHUNK), jnp.int32),
        pltpu.VMEM((CHUNK, D), jnp.float32),
        pltpu.VMEM((CHUNK, D), jnp.float32),
        pltpu.VMEM_SHARED((NP_, D), jnp.float32),
        pltpu.SemaphoreType.DMA,
        pltpu.SemaphoreType.DMA,
    ],
)
def _agg(x_hbm, src_hbm, dst_hbm, zrow_hbm, out_hbm,
         src_v, dst_v, b0, b1, acc_sh, s0, s1):
  c = lax.axis_index("c")
  s = lax.axis_index("s")
  wid = c * NS + s
  r0 = s * RPT

  def g_start(jc, buf, sem):
    pltpu.async_copy(x_hbm.at[src_v.at[jc]], buf, sem)

  def g_wait(buf, sem):
    pltpu.make_async_copy(x_hbm.at[pl.ds(0, CHUNK)], buf, sem).wait()

  pltpu.sync_copy(zrow_hbm, b0)
  for k in range(RPT // WB):
    pltpu.sync_copy(b0, acc_sh.at[pl.ds(r0 + k * WB, WB)])
  plsc.subcore_barrier()

  def step(i, carry):
    c0 = 2 * i
    c1 = 2 * i + 1
    g_wait(b0, s0)
    pltpu.sync_copy(b0, acc_sh.at[dst_v.at[c0]], add=True)
    g_start(jnp.minimum(c0 + 2, IDXB - 1), b0, s0)
    g_wait(b1, s1)
    pltpu.sync_copy(b1, acc_sh.at[dst_v.at[c1]], add=True)
    g_start(jnp.minimum(c1 + 2, IDXB - 1), b1, s1)
    return carry

  for ph in range(CPW // IDXB):
    pltpu.sync_copy(src_hbm.at[wid, pl.ds(ph * IDXB, IDXB)], src_v)
    pltpu.sync_copy(dst_hbm.at[wid, pl.ds(ph * IDXB, IDXB)], dst_v)
    g_start(0, b0, s0)
    g_start(1, b1, s1)
    lax.fori_loop(0, IDXB // 2, step, 0)
    g_wait(b0, s0)
    g_wait(b1, s1)

  plsc.subcore_barrier()

  for k in range(RPT // WB):
    pltpu.sync_copy(acc_sh.at[pl.ds(r0 + k * WB, WB)], b0)
    pltpu.sync_copy(b0, out_hbm.at[c, pl.ds(r0 + k * WB, WB)])


@pl.kernel(
    out_type=jax.ShapeDtypeStruct((NC, NP_, DEGW), jnp.float32),
    mesh=_MESH,
    scratch_types=[
        pltpu.VMEM((CPW, CHUNK), jnp.int32),
        pltpu.VMEM((CHUNK, DEGW), jnp.float32),
        pltpu.VMEM_SHARED((NP_, DEGW), jnp.float32),
        pltpu.SemaphoreType.DMA,
    ],
)
def _deg(dst_hbm, zdeg_hbm, ones_hbm, deg_out_hbm, dst_v, small_v, deg_sh, sem):
  c = lax.axis_index("c")
  s = lax.axis_index("s")
  wid = c * NS + s
  r0 = s * RPT

  pltpu.sync_copy(dst_hbm.at[wid], dst_v)
  pltpu.sync_copy(zdeg_hbm, small_v)
  for k in range(RPT // WB):
    pltpu.sync_copy(small_v, deg_sh.at[pl.ds(r0 + k * WB, WB)])
  pltpu.sync_copy(ones_hbm, small_v)
  plsc.subcore_barrier()

  def step(i, carry):
    for t in range(DEG_Q):
      pltpu.async_copy(small_v, deg_sh.at[dst_v.at[DEG_Q * i + t]], sem,
                       add=True)
    for t in range(DEG_Q):
      pltpu.make_async_copy(small_v, deg_sh.at[pl.ds(0, CHUNK)], sem).wait()
    return carry

  lax.fori_loop(0, CPW // DEG_Q, step, 0)
  plsc.subcore_barrier()

  for k in range(RPT // WB):
    pltpu.sync_copy(deg_sh.at[pl.ds(r0 + k * WB, WB)], small_v)
    pltpu.sync_copy(small_v, deg_out_hbm.at[c, pl.ds(r0 + k * WB, WB)])


def _dense(part, degp, xin, w_l, w_r, b, do_relu):
  rows = 1000

  def body(p_ref, d_ref, x_ref, wl_ref, wr_ref, b_ref, o_ref):
    agg = p_ref[0] + p_ref[1]
    deg = d_ref[0] + d_ref[1]
    degc = jnp.max(deg, axis=1, keepdims=True)
    mean = agg / jnp.maximum(degc, 1.0)
    acc = jnp.dot(mean, wl_ref[...], preferred_element_type=jnp.float32)
    acc = acc + jnp.dot(x_ref[...], wr_ref[...], preferred_element_type=jnp.float32)
    acc = acc + b_ref[...]
    if do_relu:
      acc = jnp.maximum(acc, 0.0)
    o_ref[...] = acc

  return pl.pallas_call(
      body,
      grid=(N // rows,),
      in_specs=[
          pl.BlockSpec((NC, rows, D), lambda i: (0, i, 0)),
          pl.BlockSpec((NC, rows, DEGW), lambda i: (0, i, 0)),
          pl.BlockSpec((rows, D), lambda i: (i, 0)),
          pl.BlockSpec((D, D), lambda i: (0, 0)),
          pl.BlockSpec((D, D), lambda i: (0, 0)),
          pl.BlockSpec((1, D), lambda i: (0, 0)),
      ],
      out_specs=pl.BlockSpec((rows, D), lambda i: (i, 0)),
      out_shape=jax.ShapeDtypeStruct((N, D), jnp.float32),
  )(part, degp, xin, w_l, w_r, b.reshape(1, D))


def kernel(x, edge_index, W1_l, W1_r, b1, W2_l, W2_r, b2):
  src = edge_index[0].astype(jnp.int32)
  dst = edge_index[1].astype(jnp.int32)
  pad = PADE - E
  srcp = jnp.concatenate([src, jnp.zeros((pad,), jnp.int32)])
  dstp = jnp.concatenate([dst, jnp.full((pad,), DUMMY, jnp.int32)])
  src3 = srcp.reshape(NW, CPW, CHUNK)
  dst3 = dstp.reshape(NW, CPW, CHUNK)
  zrow = jnp.zeros((WB, D), jnp.float32)
  ones = jnp.ones((CHUNK, DEGW), jnp.float32)

  degp = _deg(dst3, zrow, ones)
  part1 = _agg(x, src3, dst3, zrow)
  h = _dense(part1, degp, x, W1_l, W1_r, b1, True)
  part2 = _agg(h, src3, dst3, zrow)
  out = _dense(part2, degp, h, W2_l, W2_r, b2, False)
  return out


# speedup vs baseline: 3.2227x; 3.2227x over previous
import jax
import jax.numpy as jnp
from jax import lax
from jax.experimental import pallas as pl
from jax.experimental.pallas import tpu as pltpu
from jax.experimental.pallas import tpu_sc as plsc

N = 10000
E = 320000
D = 128
NC = 2
NS = 16
NW = NC * NS
CHUNK = 128
CPW = 80
PADE = NW * CPW * CHUNK
NP_ = 10240
DUMMY = N
RPT = NP_ // NS
WB = 128
DEGW = 128
DEG_Q = 16
IDXB = 40

_MESH = plsc.VectorSubcoreMesh(
    core_axis_name="c", subcore_axis_name="s", num_cores=NC, num_subcores=NS
)


@pl.kernel(
    out_type=jax.ShapeDtypeStruct((NC, NP_, D), jnp.float32),
    mesh=_MESH,
    scratch_types=[
        pltpu.VMEM((IDXB, CHUNK), jnp.int32),
        pltpu.VMEM((IDXB, CHUNK), jnp.int32),
        pltpu.VMEM((CHUNK, D), jnp.float32),
        pltpu.VMEM((CHUNK, D), jnp.float32),
        pltpu.VMEM_SHARED((NP_, D), jnp.float32),
        pltpu.SemaphoreType.DMA,
        pltpu.SemaphoreType.DMA,
    ],
)
def _agg(x_hbm, src_hbm, dst_hbm, zrow_hbm, out_hbm,
         src_v, dst_v, b0, b1, acc_sh, s0, s1):
  c = lax.axis_index("c")
  s = lax.axis_index("s")
  wid = c * NS + s
  r0 = s * RPT

  def g_start(jc, buf, sem):
    pltpu.async_copy(x_hbm.at[src_v.at[jc]], buf, sem)

  def g_wait(buf, sem):
    pltpu.make_async_copy(x_hbm.at[pl.ds(0, CHUNK)], buf, sem).wait()

  pltpu.sync_copy(zrow_hbm, b0)
  for k in range(RPT // WB):
    pltpu.async_copy(b0, acc_sh.at[pl.ds(r0 + k * WB, WB)], s0)
  for k in range(RPT // WB):
    pltpu.make_async_copy(b0, acc_sh.at[pl.ds(0, WB)], s0).wait()
  plsc.subcore_barrier()

  def step(i, carry):
    c0 = 2 * i
    c1 = 2 * i + 1
    g_wait(b0, s0)
    pltpu.sync_copy(b0, acc_sh.at[dst_v.at[c0]], add=True)

    @pl.when(c0 + 2 < IDXB)
    def _():
      g_start(c0 + 2, b0, s0)

    g_wait(b1, s1)
    pltpu.sync_copy(b1, acc_sh.at[dst_v.at[c1]], add=True)

    @pl.when(c1 + 2 < IDXB)
    def _():
      g_start(c1 + 2, b1, s1)

    return carry

  for ph in range(CPW // IDXB):
    pltpu.sync_copy(src_hbm.at[wid, pl.ds(ph * IDXB, IDXB)], src_v)
    pltpu.sync_copy(dst_hbm.at[wid, pl.ds(ph * IDXB, IDXB)], dst_v)
    g_start(0, b0, s0)
    g_start(1, b1, s1)
    lax.fori_loop(0, IDXB // 2, step, 0)

  plsc.subcore_barrier()

  bufs = (b0, b1)
  sems = (s0, s1)
  for k in range(RPT // WB):
    b, sm = bufs[k % 2], sems[k % 2]
    if k >= 2:
      pltpu.make_async_copy(b, out_hbm.at[c, pl.ds(0, WB)], sm).wait()
    pltpu.sync_copy(acc_sh.at[pl.ds(r0 + k * WB, WB)], b)
    pltpu.async_copy(b, out_hbm.at[c, pl.ds(r0 + k * WB, WB)], sm)
  for k in (RPT // WB - 2, RPT // WB - 1):
    b, sm = bufs[k % 2], sems[k % 2]
    pltpu.make_async_copy(b, out_hbm.at[c, pl.ds(0, WB)], sm).wait()


@pl.kernel(
    out_type=jax.ShapeDtypeStruct((NC, NP_, DEGW), jnp.float32),
    mesh=_MESH,
    scratch_types=[
        pltpu.VMEM((CPW, CHUNK), jnp.int32),
        pltpu.VMEM((CHUNK, DEGW), jnp.float32),
        pltpu.VMEM_SHARED((NP_, DEGW), jnp.float32),
        pltpu.SemaphoreType.DMA,
    ],
)
def _deg(dst_hbm, zdeg_hbm, ones_hbm, deg_out_hbm, dst_v, small_v, deg_sh, sem):
  c = lax.axis_index("c")
  s = lax.axis_index("s")
  wid = c * NS + s
  r0 = s * RPT

  pltpu.sync_copy(dst_hbm.at[wid], dst_v)
  pltpu.sync_copy(zdeg_hbm, small_v)
  for k in range(RPT // WB):
    pltpu.async_copy(small_v, deg_sh.at[pl.ds(r0 + k * WB, WB)], sem)
  for k in range(RPT // WB):
    pltpu.make_async_copy(small_v, deg_sh.at[pl.ds(0, WB)], sem).wait()
  pltpu.sync_copy(ones_hbm, small_v)
  plsc.subcore_barrier()

  def step(i, carry):
    for t in range(DEG_Q):
      pltpu.async_copy(small_v, deg_sh.at[dst_v.at[DEG_Q * i + t]], sem,
                       add=True)
    for t in range(DEG_Q):
      pltpu.make_async_copy(small_v, deg_sh.at[pl.ds(0, CHUNK)], sem).wait()
    return carry

  lax.fori_loop(0, CPW // DEG_Q, step, 0)
  plsc.subcore_barrier()

  for k in range(RPT // WB):
    pltpu.sync_copy(deg_sh.at[pl.ds(r0 + k * WB, WB)], small_v)
    pltpu.sync_copy(small_v, deg_out_hbm.at[c, pl.ds(r0 + k * WB, WB)])


def _dense(part, degp, xin, w_l, w_r, b, do_relu):
  rows = 1000

  def body(p_ref, d_ref, x_ref, wl_ref, wr_ref, b_ref, o_ref):
    agg = p_ref[0] + p_ref[1]
    deg = d_ref[0] + d_ref[1]
    degc = jnp.max(deg, axis=1, keepdims=True)
    mean = agg / jnp.maximum(degc, 1.0)
    acc = jnp.dot(mean, wl_ref[...], preferred_element_type=jnp.float32)
    acc = acc + jnp.dot(x_ref[...], wr_ref[...], preferred_element_type=jnp.float32)
    acc = acc + b_ref[...]
    if do_relu:
      acc = jnp.maximum(acc, 0.0)
    o_ref[...] = acc

  return pl.pallas_call(
      body,
      grid=(N // rows,),
      in_specs=[
          pl.BlockSpec((NC, rows, D), lambda i: (0, i, 0)),
          pl.BlockSpec((NC, rows, DEGW), lambda i: (0, i, 0)),
          pl.BlockSpec((rows, D), lambda i: (i, 0)),
          pl.BlockSpec((D, D), lambda i: (0, 0)),
          pl.BlockSpec((D, D), lambda i: (0, 0)),
          pl.BlockSpec((1, D), lambda i: (0, 0)),
      ],
      out_specs=pl.BlockSpec((rows, D), lambda i: (i, 0)),
      out_shape=jax.ShapeDtypeStruct((N, D), jnp.float32),
  )(part, degp, xin, w_l, w_r, b.reshape(1, D))


def kernel(x, edge_index, W1_l, W1_r, b1, W2_l, W2_r, b2):
  src = edge_index[0].astype(jnp.int32)
  dst = edge_index[1].astype(jnp.int32)
  pad = PADE - E
  pad_src = jnp.arange(pad, dtype=jnp.int32) % N
  pad_dst = DUMMY + jnp.arange(pad, dtype=jnp.int32) % (NP_ - N)
  srcp = jnp.concatenate([src, pad_src])
  dstp = jnp.concatenate([dst, pad_dst])
  src3 = srcp.reshape(NW, CPW, CHUNK)
  dst3 = dstp.reshape(NW, CPW, CHUNK)
  zrow = jnp.zeros((WB, D), jnp.float32)
  zdeg = jnp.zeros((CHUNK, DEGW), jnp.float32)
  ones = jnp.ones((CHUNK, DEGW), jnp.float32)

  degp = _deg(dst3, zdeg, ones)
  part1 = _agg(x, src3, dst3, zrow)
  h = _dense(part1, degp, x, W1_l, W1_r, b1, True)
  part2 = _agg(h, src3, dst3, zrow)
  out = _dense(part2, degp, h, W2_l, W2_r, b2, False)
  return out


# speedup vs baseline: 3.2408x; 1.0056x over previous
import jax
import jax.numpy as jnp
from jax import lax
from jax.experimental import pallas as pl
from jax.experimental.pallas import tpu as pltpu
from jax.experimental.pallas import tpu_sc as plsc

N = 10000
E = 320000
D = 128
NC = 2
NS = 16
NW = NC * NS
CHUNK = 128
CPW = 80
PADE = NW * CPW * CHUNK
NP_ = 10240
DUMMY = N
RPT = NP_ // NS
WB = 128
DEGW = 128
DEG_Q = 20
IDXB = 40

_MESH = plsc.VectorSubcoreMesh(
    core_axis_name="c", subcore_axis_name="s", num_cores=NC, num_subcores=NS
)


@pl.kernel(
    out_type=jax.ShapeDtypeStruct((NC, NP_, D), jnp.float32),
    mesh=_MESH,
    scratch_types=[
        pltpu.VMEM((IDXB, CHUNK), jnp.int32),
        pltpu.VMEM((IDXB, CHUNK), jnp.int32),
        pltpu.VMEM((CHUNK, D), jnp.float32),
        pltpu.VMEM((CHUNK, D), jnp.float32),
        pltpu.VMEM_SHARED((NP_, D), jnp.float32),
        pltpu.SemaphoreType.DMA,
        pltpu.SemaphoreType.DMA,
    ],
)
def _agg(x_hbm, src_hbm, dst_hbm, zrow_hbm, out_hbm,
         src_v, dst_v, b0, b1, acc_sh, s0, s1):
  c = lax.axis_index("c")
  s = lax.axis_index("s")
  wid = c * NS + s
  r0 = s * RPT

  def g_start(jc, buf, sem):
    pltpu.async_copy(x_hbm.at[src_v.at[jc]], buf, sem)

  def g_wait(buf, sem):
    pltpu.make_async_copy(x_hbm.at[pl.ds(0, CHUNK)], buf, sem).wait()

  pltpu.sync_copy(zrow_hbm, b0)
  for k in range(RPT // WB):
    pltpu.async_copy(b0, acc_sh.at[pl.ds(r0 + k * WB, WB)], s0)
  for k in range(RPT // WB):
    pltpu.make_async_copy(b0, acc_sh.at[pl.ds(0, WB)], s0).wait()
  plsc.subcore_barrier()

  def step(i, carry):
    c0 = 2 * i
    c1 = 2 * i + 1
    g_wait(b0, s0)
    pltpu.sync_copy(b0, acc_sh.at[dst_v.at[c0]], add=True)

    @pl.when(c0 + 2 < IDXB)
    def _():
      g_start(c0 + 2, b0, s0)

    g_wait(b1, s1)
    pltpu.sync_copy(b1, acc_sh.at[dst_v.at[c1]], add=True)

    @pl.when(c1 + 2 < IDXB)
    def _():
      g_start(c1 + 2, b1, s1)

    return carry

  for ph in range(CPW // IDXB):
    pltpu.sync_copy(src_hbm.at[wid, pl.ds(ph * IDXB, IDXB)], src_v)
    pltpu.sync_copy(dst_hbm.at[wid, pl.ds(ph * IDXB, IDXB)], dst_v)
    g_start(0, b0, s0)
    g_start(1, b1, s1)
    lax.fori_loop(0, IDXB // 2, step, 0)

  plsc.subcore_barrier()

  bufs = (b0, b1)
  sems = (s0, s1)
  for k in range(RPT // WB):
    b, sm = bufs[k % 2], sems[k % 2]
    if k >= 2:
      pltpu.make_async_copy(b, out_hbm.at[c, pl.ds(0, WB)], sm).wait()
    pltpu.sync_copy(acc_sh.at[pl.ds(r0 + k * WB, WB)], b)
    pltpu.async_copy(b, out_hbm.at[c, pl.ds(r0 + k * WB, WB)], sm)
  for k in (RPT // WB - 2, RPT // WB - 1):
    b, sm = bufs[k % 2], sems[k % 2]
    pltpu.make_async_copy(b, out_hbm.at[c, pl.ds(0, WB)], sm).wait()


@pl.kernel(
    out_type=jax.ShapeDtypeStruct((NC, NP_, DEGW), jnp.float32),
    mesh=_MESH,
    scratch_types=[
        pltpu.VMEM((CPW, CHUNK), jnp.int32),
        pltpu.VMEM((CHUNK, DEGW), jnp.float32),
        pltpu.VMEM((CHUNK, DEGW), jnp.float32),
        pltpu.VMEM_SHARED((NP_, DEGW), jnp.float32),
        pltpu.SemaphoreType.DMA,
        pltpu.SemaphoreType.DMA,
    ],
)
def _deg(dst_hbm, zdeg_hbm, ones_hbm, deg_out_hbm, dst_v, small_v, small2,
         deg_sh, sem, sem2):
  c = lax.axis_index("c")
  s = lax.axis_index("s")
  wid = c * NS + s
  r0 = s * RPT

  pltpu.sync_copy(dst_hbm.at[wid], dst_v)
  pltpu.sync_copy(zdeg_hbm, small_v)
  for k in range(RPT // WB):
    pltpu.async_copy(small_v, deg_sh.at[pl.ds(r0 + k * WB, WB)], sem)
  for k in range(RPT // WB):
    pltpu.make_async_copy(small_v, deg_sh.at[pl.ds(0, WB)], sem).wait()
  pltpu.sync_copy(ones_hbm, small_v)
  plsc.subcore_barrier()

  def step(i, carry):
    for t in range(DEG_Q):
      pltpu.async_copy(small_v, deg_sh.at[dst_v.at[DEG_Q * i + t]], sem,
                       add=True)
    for t in range(DEG_Q):
      pltpu.make_async_copy(small_v, deg_sh.at[pl.ds(0, CHUNK)], sem).wait()
    return carry

  lax.fori_loop(0, CPW // DEG_Q, step, 0)
  plsc.subcore_barrier()

  bufs = (small_v, small2)
  sems = (sem, sem2)
  for k in range(RPT // WB):
    b, sm = bufs[k % 2], sems[k % 2]
    if k >= 2:
      pltpu.make_async_copy(b, deg_out_hbm.at[c, pl.ds(0, WB)], sm).wait()
    pltpu.sync_copy(deg_sh.at[pl.ds(r0 + k * WB, WB)], b)
    pltpu.async_copy(b, deg_out_hbm.at[c, pl.ds(r0 + k * WB, WB)], sm)
  for k in (RPT // WB - 2, RPT // WB - 1):
    b, sm = bufs[k % 2], sems[k % 2]
    pltpu.make_async_copy(b, deg_out_hbm.at[c, pl.ds(0, WB)], sm).wait()


def _dense(part, degp, xin, w_l, w_r, b, do_relu):
  rows = 1000

  def body(p_ref, d_ref, x_ref, wl_ref, wr_ref, b_ref, o_ref):
    agg = p_ref[0] + p_ref[1]
    deg = d_ref[0] + d_ref[1]
    degc = jnp.max(deg, axis=1, keepdims=True)
    mean = agg / jnp.maximum(degc, 1.0)
    acc = jnp.dot(mean, wl_ref[...], preferred_element_type=jnp.float32)
    acc = acc + jnp.dot(x_ref[...], wr_ref[...], preferred_element_type=jnp.float32)
    acc = acc + b_ref[...]
    if do_relu:
      acc = jnp.maximum(acc, 0.0)
    o_ref[...] = acc

  return pl.pallas_call(
      body,
      grid=(N // rows,),
      in_specs=[
          pl.BlockSpec((NC, rows, D), lambda i: (0, i, 0)),
          pl.BlockSpec((NC, rows, DEGW), lambda i: (0, i, 0)),
          pl.BlockSpec((rows, D), lambda i: (i, 0)),
          pl.BlockSpec((D, D), lambda i: (0, 0)),
          pl.BlockSpec((D, D), lambda i: (0, 0)),
          pl.BlockSpec((1, D), lambda i: (0, 0)),
      ],
      out_specs=pl.BlockSpec((rows, D), lambda i: (i, 0)),
      out_shape=jax.ShapeDtypeStruct((N, D), jnp.float32),
  )(part, degp, xin, w_l, w_r, b.reshape(1, D))


def kernel(x, edge_index, W1_l, W1_r, b1, W2_l, W2_r, b2):
  src = edge_index[0].astype(jnp.int32)
  dst = edge_index[1].astype(jnp.int32)
  pad = PADE - E
  pad_src = jnp.arange(pad, dtype=jnp.int32) % N
  pad_dst = DUMMY + jnp.arange(pad, dtype=jnp.int32) % (NP_ - N)
  srcp = jnp.concatenate([src, pad_src])
  dstp = jnp.concatenate([dst, pad_dst])
  src3 = srcp.reshape(NW, CPW, CHUNK)
  dst3 = dstp.reshape(NW, CPW, CHUNK)
  zrow = jnp.zeros((WB, D), jnp.float32)
  zdeg = jnp.zeros((CHUNK, DEGW), jnp.float32)
  ones = jnp.ones((CHUNK, DEGW), jnp.float32)

  degp = _deg(dst3, zdeg, ones)
  part1 = _agg(x, src3, dst3, zrow)
  h = _dense(part1, degp, x, W1_l, W1_r, b1, True)
  part2 = _agg(h, src3, dst3, zrow)
  out = _dense(part2, degp, h, W2_l, W2_r, b2, False)
  return out
